# baseline (speedup 1.0000x reference)
import functools

import jax
import jax.numpy as jnp
from jax import lax
from jax.experimental import pallas as pl
from jax.experimental.pallas import tpu as pltpu
from jax.experimental.pallas import tpu_sc as plsc

NUM_FACTORS = 32
BATCH = 16384
H0, H1, H2 = 64, 32, 16

NC, NS = 2, 16
NW = NC * NS
BPW = BATCH // NW
CH = 128
NCH = BPW // CH

BLK = 2048


def _sc_gather(uid2d, iid2d, P, Q, U, V):
    mesh = plsc.VectorSubcoreMesh(core_axis_name="c", subcore_axis_name="s")
    out_t = tuple(jax.ShapeDtypeStruct((BATCH, NUM_FACTORS), jnp.float32)
                  for _ in range(4))

    @functools.partial(
        pl.kernel, mesh=mesh, out_type=out_t,
        compiler_params=pltpu.CompilerParams(use_tc_tiling_on_sc=False),
        scratch_types=[
            pltpu.VMEM((NCH, CH), jnp.int32),
            pltpu.VMEM((NCH, CH), jnp.int32),
            pltpu.VMEM((BPW, NUM_FACTORS), jnp.float32),
            pltpu.VMEM((BPW, NUM_FACTORS), jnp.float32),
            pltpu.VMEM((BPW, NUM_FACTORS), jnp.float32),
            pltpu.VMEM((BPW, NUM_FACTORS), jnp.float32),
            pltpu.SemaphoreType.DMA,
        ],
    )
    def gather_kernel(uid_hbm, iid_hbm, p_hbm, q_hbm, u_hbm, v_hbm,
                      p_out, q_out, u_out, v_out,
                      uidx, iidx, pr, qr, ur, vr, sem):
        wid = lax.axis_index("s") * NC + lax.axis_index("c")
        row0 = wid * NCH
        pltpu.sync_copy(uid_hbm.at[pl.ds(row0, NCH)], uidx)
        pltpu.sync_copy(iid_hbm.at[pl.ds(row0, NCH)], iidx)
        copies = []
        for j in range(NCH):
            sl = pl.ds(j * CH, CH)
            copies.append(pltpu.async_copy(p_hbm.at[uidx.at[j]], pr.at[sl], sem))
            copies.append(pltpu.async_copy(u_hbm.at[uidx.at[j]], ur.at[sl], sem))
            copies.append(pltpu.async_copy(q_hbm.at[iidx.at[j]], qr.at[sl], sem))
            copies.append(pltpu.async_copy(v_hbm.at[iidx.at[j]], vr.at[sl], sem))
        for c in copies:
            c.wait()
        base = wid * BPW
        pltpu.sync_copy(pr, p_out.at[pl.ds(base, BPW)])
        pltpu.sync_copy(qr, q_out.at[pl.ds(base, BPW)])
        pltpu.sync_copy(ur, u_out.at[pl.ds(base, BPW)])
        pltpu.sync_copy(vr, v_out.at[pl.ds(base, BPW)])

    return gather_kernel(uid2d, iid2d, P, Q, U, V)


def _mlp_body(p_ref, q_ref, u_ref, v_ref, w0_ref, b0_ref, w1_ref, b1_ref,
              w2_ref, b2_ref, wp_ref, out_ref):
    hi = lax.Precision.HIGHEST
    gmf = p_ref[...] * q_ref[...]
    w0 = w0_ref[...]
    h = (jnp.dot(u_ref[...], w0[:NUM_FACTORS], precision=hi)
         + jnp.dot(v_ref[...], w0[NUM_FACTORS:], precision=hi) + b0_ref[...])
    h = jnp.maximum(h, 0.0)
    h = jnp.maximum(jnp.dot(h, w1_ref[...], precision=hi) + b1_ref[...], 0.0)
    h = jnp.maximum(jnp.dot(h, w2_ref[...], precision=hi) + b2_ref[...], 0.0)
    wp = wp_ref[...]
    out_ref[...] = (jnp.dot(gmf, wp[:NUM_FACTORS], precision=hi)
                    + jnp.dot(h, wp[NUM_FACTORS:], precision=hi))


def _mlp(p, q, u, v, W0, b0, W1, b1, W2, b2, Wp):
    n_blk = BATCH // BLK
    row_spec = lambda d: pl.BlockSpec((BLK, d), lambda i: (i, 0))
    full = lambda s: pl.BlockSpec(s, lambda i: (0, 0))
    return pl.pallas_call(
        _mlp_body,
        grid=(n_blk,),
        in_specs=[
            row_spec(NUM_FACTORS), row_spec(NUM_FACTORS),
            row_spec(NUM_FACTORS), row_spec(NUM_FACTORS),
            full((2 * NUM_FACTORS, H0)), full((1, H0)),
            full((H0, H1)), full((1, H1)),
            full((H1, H2)), full((1, H2)),
            full((H2 + NUM_FACTORS, 1)),
        ],
        out_specs=pl.BlockSpec((BLK, 1), lambda i: (i, 0)),
        out_shape=jax.ShapeDtypeStruct((BATCH, 1), jnp.float32),
    )(p, q, u, v, W0, b0.reshape(1, H0), W1, b1.reshape(1, H1),
      W2, b2.reshape(1, H2), Wp)


def kernel(user_id, item_id, P, Q, U, V, W0, b0, W1, b1, W2, b2, Wp):
    uid2d = user_id.astype(jnp.int32).reshape(BATCH // CH, CH)
    iid2d = item_id.astype(jnp.int32).reshape(BATCH // CH, CH)
    p, q, u, v = _sc_gather(uid2d, iid2d, P, Q, U, V)
    return _mlp(p, q, u, v, W0, b0, W1, b1, W2, b2, Wp)
